# Initial kernel scaffold; baseline (speedup 1.0000x reference)
import jax
import jax.numpy as jnp
from jax.experimental import pallas as pl


def kernel(x, lengths, T_logits, f_logits):
    raise NotImplementedError("write your pallas kernel here")


# speedup vs baseline: 6.1802x; 6.1802x over previous
import jax
import jax.numpy as jnp
from jax.experimental import pallas as pl
from jax.experimental.pallas import tpu as pltpu

_Q = 128
_A = 64
_B = 16
_L = 512


def _fwd_kernel(x_ref, len_ref, tl_ref, f_ref, out_ref, tp_ref):
    logits = tl_ref[...]
    m = jnp.max(logits, axis=-1, keepdims=True)
    e = jnp.exp(logits - m)
    tp_ref[...] = e / jnp.sum(e, axis=-1, keepdims=True)

    fl = f_ref[...]
    fe = jnp.exp(fl - jnp.max(fl))
    fprob = fe / jnp.sum(fe)

    lane = jax.lax.broadcasted_iota(jnp.int32, (_B, _Q), 1)
    alpha0 = jnp.where(lane == 0, 1.0, 0.0).astype(jnp.float32)
    logscale0 = jnp.zeros((_B, _Q), jnp.float32)
    lens = len_ref[...]

    def body(t, carry):
        alpha, logscale = carry
        rows = []
        for b in range(_B):
            sym = x_ref[b, t]
            tb = tp_ref[sym]
            rows.append(jax.lax.dot_general(
                alpha[b:b + 1, :], tb, (((1,), (0,)), ((), ())),
                preferred_element_type=jnp.float32,
                precision=jax.lax.Precision.HIGHEST))
        new = jnp.concatenate(rows, axis=0)
        s = jnp.sum(new, axis=1, keepdims=True)
        mask = lens > t
        alpha = jnp.where(mask, new / s, alpha)
        logscale = logscale + jnp.where(mask, jnp.log(s), 0.0)
        return alpha, logscale

    alpha, logscale = jax.lax.fori_loop(0, _L, body, (alpha0, logscale0))
    rs = jnp.sum(alpha * fprob, axis=1, keepdims=True)
    out_ref[...] = logscale + jnp.log(rs)


def kernel(x, lengths, T_logits, f_logits):
    tl = jnp.transpose(T_logits, (1, 0, 2))
    lenb = jnp.broadcast_to(lengths.astype(jnp.int32)[:, None], (_B, _Q))
    fl = f_logits.reshape(1, _Q)
    out = pl.pallas_call(
        _fwd_kernel,
        out_shape=jax.ShapeDtypeStruct((_B, _Q), jnp.float32),
        in_specs=[
            pl.BlockSpec(memory_space=pltpu.SMEM),
            pl.BlockSpec(memory_space=pltpu.VMEM),
            pl.BlockSpec(memory_space=pltpu.VMEM),
            pl.BlockSpec(memory_space=pltpu.VMEM),
        ],
        out_specs=pl.BlockSpec(memory_space=pltpu.VMEM),
        scratch_shapes=[pltpu.VMEM((_A, _Q, _Q), jnp.float32)],
    )(x.astype(jnp.int32), lenb, tl, fl)
    return out[:, 0]


# speedup vs baseline: 17.5958x; 2.8471x over previous
import jax
import jax.numpy as jnp
from jax.experimental import pallas as pl
from jax.experimental.pallas import tpu as pltpu

_Q = 128
_A = 64
_B = 16
_L = 512


def _fwd_kernel(x_ref, len_ref, tl_ref, f_ref, out_ref, tp_ref):
    logits = tl_ref[...]
    m = jnp.max(logits, axis=-1, keepdims=True)
    e = jnp.exp(logits - m)
    tp_ref[...] = e / jnp.sum(e, axis=-1, keepdims=True)

    fl = f_ref[...]
    fe = jnp.exp(fl - jnp.max(fl))
    fprob = fe / jnp.sum(fe)

    lane = jax.lax.broadcasted_iota(jnp.int32, (_B, _Q), 1)
    alpha0 = jnp.where(lane == 0, 1.0, 0.0).astype(jnp.float32)
    logscale0 = jnp.zeros((_B, _Q), jnp.float32)
    lens = len_ref[...]

    def body(t, carry):
        alpha, logscale = carry
        rows = []
        for b in range(_B):
            sym = x_ref[b, t]
            tb = tp_ref[sym]
            rows.append(jax.lax.dot_general(
                alpha[b:b + 1, :], tb, (((1,), (0,)), ((), ())),
                preferred_element_type=jnp.float32,
                precision=jax.lax.Precision.DEFAULT))
        new = jnp.concatenate(rows, axis=0)
        s = jnp.sum(new, axis=1, keepdims=True)
        mask = lens > t
        alpha = jnp.where(mask, new / s, alpha)
        logscale = logscale + jnp.where(mask, jnp.log(s), 0.0)
        return alpha, logscale

    alpha, logscale = jax.lax.fori_loop(0, _L, body, (alpha0, logscale0))
    rs = jnp.sum(alpha * fprob, axis=1, keepdims=True)
    out_ref[...] = logscale + jnp.log(rs)


def kernel(x, lengths, T_logits, f_logits):
    tl = jnp.transpose(T_logits, (1, 0, 2))
    lenb = jnp.broadcast_to(lengths.astype(jnp.int32)[:, None], (_B, _Q))
    fl = f_logits.reshape(1, _Q)
    out = pl.pallas_call(
        _fwd_kernel,
        out_shape=jax.ShapeDtypeStruct((_B, _Q), jnp.float32),
        in_specs=[
            pl.BlockSpec(memory_space=pltpu.SMEM),
            pl.BlockSpec(memory_space=pltpu.VMEM),
            pl.BlockSpec(memory_space=pltpu.VMEM),
            pl.BlockSpec(memory_space=pltpu.VMEM),
        ],
        out_specs=pl.BlockSpec(memory_space=pltpu.VMEM),
        scratch_shapes=[pltpu.VMEM((_A, _Q, _Q), jnp.float32)],
    )(x.astype(jnp.int32), lenb, tl, fl)
    return out[:, 0]


# speedup vs baseline: 23.4098x; 1.3304x over previous
import jax
import jax.numpy as jnp
from jax.experimental import pallas as pl
from jax.experimental.pallas import tpu as pltpu

_Q = 128
_A = 64
_B = 16
_L = 512
_K = 8


def _fwd_kernel(x_ref, len_ref, tl_ref, f_ref, out_ref, tp_ref):
    logits = tl_ref[...]
    m = jnp.max(logits, axis=-1, keepdims=True)
    e = jnp.exp(logits - m)
    tp_ref[...] = (e / jnp.sum(e, axis=-1, keepdims=True)).astype(jnp.bfloat16)

    fl = f_ref[...]
    fe = jnp.exp(fl - jnp.max(fl))
    fprob = fe / jnp.sum(fe)

    lane = jax.lax.broadcasted_iota(jnp.int32, (_B, _Q), 1)
    alpha0 = jnp.where(lane == 0, 1.0, 0.0).astype(jnp.float32)
    logscale0 = jnp.zeros((_B, _Q), jnp.float32)
    lens = len_ref[...]

    def body(i, carry):
        alpha, logscale = carry
        for k in range(_K):
            t = i * _K + k
            ab = alpha.astype(jnp.bfloat16)
            rows = []
            for b in range(_B):
                sym = x_ref[b, t]
                tb = tp_ref[sym]
                rows.append(jax.lax.dot_general(
                    ab[b:b + 1, :], tb, (((1,), (0,)), ((), ())),
                    preferred_element_type=jnp.float32))
            new = jnp.concatenate(rows, axis=0)
            alpha = jnp.where(lens > t, new, alpha)
        s = jnp.sum(alpha, axis=1, keepdims=True)
        return alpha / s, logscale + jnp.log(s)

    alpha, logscale = jax.lax.fori_loop(0, _L // _K, body, (alpha0, logscale0))
    rs = jnp.sum(alpha * fprob, axis=1, keepdims=True)
    out_ref[...] = logscale + jnp.log(rs)


def kernel(x, lengths, T_logits, f_logits):
    tl = jnp.transpose(T_logits, (1, 0, 2))
    lenb = jnp.broadcast_to(lengths.astype(jnp.int32)[:, None], (_B, _Q))
    fl = f_logits.reshape(1, _Q)
    out = pl.pallas_call(
        _fwd_kernel,
        out_shape=jax.ShapeDtypeStruct((_B, _Q), jnp.float32),
        in_specs=[
            pl.BlockSpec(memory_space=pltpu.SMEM),
            pl.BlockSpec(memory_space=pltpu.VMEM),
            pl.BlockSpec(memory_space=pltpu.VMEM),
            pl.BlockSpec(memory_space=pltpu.VMEM),
        ],
        out_specs=pl.BlockSpec(memory_space=pltpu.VMEM),
        scratch_shapes=[pltpu.VMEM((_A, _Q, _Q), jnp.bfloat16)],
    )(x.astype(jnp.int32), lenb, tl, fl)
    return out[:, 0]


# speedup vs baseline: 30.6168x; 1.3079x over previous
import jax
import jax.numpy as jnp
from jax.experimental import pallas as pl
from jax.experimental.pallas import tpu as pltpu

_Q = 128
_A = 64
_B = 16
_L = 512
_K = 4


def _fwd_kernel(x_ref, len_ref, tl_ref, f_ref, out_ref, tp_ref):
    logits = tl_ref[...]
    m = jnp.max(logits, axis=-1, keepdims=True)
    e = jnp.exp(logits - m)
    tp_ref[...] = (e / jnp.sum(e, axis=-1, keepdims=True)).astype(jnp.bfloat16)

    fl = f_ref[...]
    fe = jnp.exp(fl - jnp.max(fl))
    fprob = fe / jnp.sum(fe)

    lane = jax.lax.broadcasted_iota(jnp.int32, (1, _Q), 1)
    row0 = jnp.where(lane == 0, 1.0, 0.0).astype(jnp.float32)
    ls0 = jnp.zeros((1, 1), jnp.float32)
    rows0 = tuple(row0 for _ in range(_B))
    ls_init = tuple(ls0 for _ in range(_B))
    lens = tuple(len_ref[b] for b in range(_B))

    def body(i, carry):
        rows, lss = carry
        rows = list(rows)
        for k in range(_K):
            t = i * _K + k
            for b in range(_B):
                sym = x_ref[b, t]
                tb = tp_ref[sym]
                new = jax.lax.dot_general(
                    rows[b].astype(jnp.bfloat16), tb,
                    (((1,), (0,)), ((), ())),
                    preferred_element_type=jnp.float32)
                rows[b] = jnp.where(t < lens[b], new, rows[b])
        new_rows, new_lss = [], []
        for b in range(_B):
            s = jnp.sum(rows[b], axis=1, keepdims=True)
            new_rows.append(rows[b] / s)
            new_lss.append(lss[b] + jnp.log(s))
        return tuple(new_rows), tuple(new_lss)

    rows, lss = jax.lax.fori_loop(0, _L // _K, body, (rows0, ls_init))
    for b in range(_B):
        rs = jnp.sum(rows[b] * fprob, axis=1, keepdims=True)
        out_ref[b:b + 1, :] = jnp.broadcast_to(lss[b] + jnp.log(rs), (1, _Q))


def kernel(x, lengths, T_logits, f_logits):
    tl = jnp.transpose(T_logits, (1, 0, 2))
    fl = f_logits.reshape(1, _Q)
    out = pl.pallas_call(
        _fwd_kernel,
        out_shape=jax.ShapeDtypeStruct((_B, _Q), jnp.float32),
        in_specs=[
            pl.BlockSpec(memory_space=pltpu.SMEM),
            pl.BlockSpec(memory_space=pltpu.SMEM),
            pl.BlockSpec(memory_space=pltpu.VMEM),
            pl.BlockSpec(memory_space=pltpu.VMEM),
        ],
        out_specs=pl.BlockSpec(memory_space=pltpu.VMEM),
        scratch_shapes=[pltpu.VMEM((_A, _Q, _Q), jnp.bfloat16)],
    )(x.astype(jnp.int32), lengths.astype(jnp.int32), tl, fl)
    return out[:, 0]


# speedup vs baseline: 30.7473x; 1.0043x over previous
import jax
import jax.numpy as jnp
from jax.experimental import pallas as pl
from jax.experimental.pallas import tpu as pltpu

_Q = 128
_A = 64
_B = 16
_L = 512
_K = 4


def _fwd_kernel(x_ref, len_ref, tl_ref, f_ref, out_ref, tp_ref):
    logits = tl_ref[...]
    m = jnp.max(logits, axis=-1, keepdims=True)
    e = jnp.exp(logits - m)
    tp_ref[...] = (e / jnp.sum(e, axis=-1, keepdims=True)).astype(jnp.bfloat16)

    fl = f_ref[...]
    fe = jnp.exp(fl - jnp.max(fl))
    fprob = fe / jnp.sum(fe)

    lane = jax.lax.broadcasted_iota(jnp.int32, (1, _Q), 1)
    row0 = jnp.where(lane == 0, 1.0, 0.0).astype(jnp.float32)
    ls0 = jnp.zeros((1, 1), jnp.int32)
    rows0 = tuple(row0 for _ in range(_B))
    ls_init = tuple(ls0 for _ in range(_B))
    lens = tuple(len_ref[b] for b in range(_B))

    def body(i, carry):
        rows, esums = carry
        rows = list(rows)
        for k in range(_K):
            t = i * _K + k
            for b in range(_B):
                sym = x_ref[b, t]
                tb = tp_ref[sym]
                new = jax.lax.dot_general(
                    rows[b].astype(jnp.bfloat16), tb,
                    (((1,), (0,)), ((), ())),
                    preferred_element_type=jnp.float32)
                rows[b] = jnp.where(t < lens[b], new, rows[b])
        new_rows, new_esums = [], []
        for b in range(_B):
            s = jnp.sum(rows[b], axis=1, keepdims=True)
            biased = jax.lax.shift_right_logical(
                jax.lax.bitcast_convert_type(s, jnp.int32), 23) & 0xFF
            scale = jax.lax.bitcast_convert_type(
                jax.lax.shift_left(254 - biased, 23), jnp.float32)
            new_rows.append(rows[b] * scale)
            new_esums.append(esums[b] + (biased - 127))
        return tuple(new_rows), tuple(new_esums)

    rows, esums = jax.lax.fori_loop(0, _L // _K, body, (rows0, ls_init))
    ln2 = 0.6931471805599453
    for b in range(_B):
        rs = jnp.sum(rows[b] * fprob, axis=1, keepdims=True)
        tot = jnp.log(rs) + esums[b].astype(jnp.float32) * ln2
        out_ref[b:b + 1, :] = jnp.broadcast_to(tot, (1, _Q))


def kernel(x, lengths, T_logits, f_logits):
    tl = jnp.transpose(T_logits, (1, 0, 2))
    fl = f_logits.reshape(1, _Q)
    out = pl.pallas_call(
        _fwd_kernel,
        out_shape=jax.ShapeDtypeStruct((_B, _Q), jnp.float32),
        in_specs=[
            pl.BlockSpec(memory_space=pltpu.SMEM),
            pl.BlockSpec(memory_space=pltpu.SMEM),
            pl.BlockSpec(memory_space=pltpu.VMEM),
            pl.BlockSpec(memory_space=pltpu.VMEM),
        ],
        out_specs=pl.BlockSpec(memory_space=pltpu.VMEM),
        scratch_shapes=[pltpu.VMEM((_A, _Q, _Q), jnp.bfloat16)],
    )(x.astype(jnp.int32), lengths.astype(jnp.int32), tl, fl)
    return out[:, 0]
